# baseline (speedup 1.0000x reference)
import functools

import jax
import jax.numpy as jnp
from jax import lax
from jax.experimental import pallas as pl
from jax.experimental.pallas import tpu as pltpu
from jax.experimental.pallas import tpu_sc as plsc

N_NODES = 10000
D = 128
E = 320000
DEPTH = 3
NC = 2
NS = 16
NW = NC * NS
EPT = E // NW + 240
CH = EPT // 128
EP = EPT * NW
NP = 10240
NSL = NP // NS
GC = 64
CHN = EPT // GC

_mesh = plsc.VectorSubcoreMesh(
    core_axis_name="c", subcore_axis_name="s", num_cores=NC, num_subcores=NS
)


def _zero_vmem_2d(ref, rows):
    zero16 = jnp.zeros((16,), jnp.float32)

    def body(r, carry):
        for k in range(8):
            ref[r, pl.ds(k * 16, 16)] = zero16
        return carry

    lax.fori_loop(0, rows, body, 0)


def _zero_vmem_1d(ref, n):
    zero16 = jnp.zeros((16,), jnp.float32)

    def body(i, carry):
        ref[pl.ds(i * 16, 16)] = zero16
        return carry

    lax.fori_loop(0, n // 16, body, 0)


@functools.partial(
    pl.kernel,
    out_type=jax.ShapeDtypeStruct((NC, NP), jnp.float32),
    mesh=_mesh,
    compiler_params=pltpu.CompilerParams(needs_layout_passes=False),
    scratch_types=[
        pltpu.VMEM((CH, 128), jnp.int32),
        pltpu.VMEM((128,), jnp.float32),
        pltpu.VMEM((NSL,), jnp.float32),
        pltpu.VMEM_SHARED((NP,), jnp.float32),
    ],
)
def _deg_kernel(row_hbm, degp_hbm, row_v, ones_v, zrow_v, deg_sh):
    cid = lax.axis_index("c")
    sid = lax.axis_index("s")
    wid = cid * NS + sid
    pltpu.sync_copy(row_hbm.at[wid], row_v)
    one16 = jnp.ones((16,), jnp.float32)
    for k in range(8):
        ones_v[pl.ds(k * 16, 16)] = one16
    _zero_vmem_1d(zrow_v, NSL)
    pltpu.sync_copy(zrow_v, deg_sh.at[pl.ds(sid * NSL, NSL)])
    plsc.subcore_barrier()

    def body(j, carry):
        pltpu.sync_copy(ones_v, deg_sh.at[row_v.at[j]], add=True)
        return carry

    lax.fori_loop(0, CH, body, 0)
    plsc.subcore_barrier()
    pltpu.sync_copy(
        deg_sh.at[pl.ds(sid * NSL, NSL)], degp_hbm.at[cid, pl.ds(sid * NSL, NSL)]
    )


@functools.partial(
    pl.kernel,
    out_type=jax.ShapeDtypeStruct((NW, CH, 128), jnp.float32),
    mesh=_mesh,
    compiler_params=pltpu.CompilerParams(needs_layout_passes=False),
    scratch_types=[
        pltpu.VMEM((CH, 128), jnp.int32),
        pltpu.VMEM((CH, 128), jnp.int32),
        pltpu.VMEM((CH, 128), jnp.float32),
        pltpu.VMEM((NP,), jnp.float32),
    ],
)
def _val_kernel(row_hbm, col_hbm, attr_hbm, dinv_hbm, val_hbm,
                row_v, col_v, val_v, dinv_v):
    cid = lax.axis_index("c")
    sid = lax.axis_index("s")
    wid = cid * NS + sid
    pltpu.sync_copy(row_hbm.at[wid], row_v)
    pltpu.sync_copy(col_hbm.at[wid], col_v)
    pltpu.sync_copy(attr_hbm.at[wid], val_v)
    pltpu.sync_copy(dinv_hbm, dinv_v)

    def val_body(j, carry):
        for k in range(8):
            s = pl.ds(k * 16, 16)
            r16 = row_v[j, s]
            c16 = col_v[j, s]
            a16 = val_v[j, s]
            dr = plsc.load_gather(dinv_v, [r16])
            dc = plsc.load_gather(dinv_v, [c16])
            val_v[j, s] = dr * a16 * dc
        return carry

    lax.fori_loop(0, CH, val_body, 0)
    pltpu.sync_copy(val_v, val_hbm.at[wid])


@functools.partial(
    pl.kernel,
    out_type=jax.ShapeDtypeStruct((NC, NP, D), jnp.float32),
    mesh=_mesh,
    compiler_params=pltpu.CompilerParams(needs_layout_passes=False),
    scratch_types=[
        pltpu.VMEM((EPT,), jnp.int32),
        pltpu.VMEM((EPT,), jnp.int32),
        pltpu.VMEM((EPT,), jnp.float32),
        pltpu.VMEM((2, GC, D), jnp.float32),
        pltpu.VMEM_SHARED((NP, D), jnp.float32),
        pltpu.SemaphoreType.DMA,
        pltpu.SemaphoreType.DMA,
    ],
)
def _spmm_kernel(
    y_hbm, row_hbm, col_hbm, valw_hbm, part_hbm,
    row_v, col_v, val_v, g2, acc_sh, sem_g, sem_s
):
    cid = lax.axis_index("c")
    sid = lax.axis_index("s")
    wid = cid * NS + sid
    pltpu.sync_copy(row_hbm.at[wid], row_v)
    pltpu.sync_copy(col_hbm.at[wid], col_v)
    pltpu.sync_copy(valw_hbm.at[wid], val_v)

    _zero_vmem_2d(g2.at[0], GC)
    for blk in range(NSL // GC):
        pltpu.sync_copy(g2.at[0], acc_sh.at[pl.ds(sid * NSL + blk * GC, GC)])
    plsc.subcore_barrier()

    def gather_desc(c, b):
        return pltpu.make_async_copy(
            y_hbm.at[col_v.at[pl.ds(c * GC, GC)]], g2.at[b], sem_g
        )

    def scatter_wait(c, b):
        pltpu.make_async_copy(
            g2.at[b], acc_sh.at[row_v.at[pl.ds(c * GC, GC)]], sem_s
        ).wait()

    gather_desc(0, 0).start()

    def chunk_body(c, carry):
        b = c % 2
        nb = (c + 1) % 2
        gather_desc(c, b).wait()

        def scale_body(e0, c2):
            vchunk = val_v[pl.ds(c * GC + e0 * 16, 16)]
            for l in range(16):
                v = vchunk[l]
                e = e0 * 16 + l
                for k in range(8):
                    s = pl.ds(k * 16, 16)
                    g2[b, e, s] = g2[b, e, s] * v
            return c2

        lax.fori_loop(0, GC // 16, scale_body, 0)
        pltpu.async_copy(
            g2.at[b], acc_sh.at[row_v.at[pl.ds(c * GC, GC)]], sem_s, add=True
        )

        @pl.when(jnp.logical_and(c >= 1, c + 1 < CHN))
        def _():
            scatter_wait(c - 1, nb)

        @pl.when(c + 1 < CHN)
        def _():
            gather_desc(c + 1, nb).start()

        return carry

    lax.fori_loop(0, CHN, chunk_body, 0)
    scatter_wait(CHN - 2, (CHN - 2) % 2)
    scatter_wait(CHN - 1, (CHN - 1) % 2)
    plsc.subcore_barrier()
    pltpu.sync_copy(
        acc_sh.at[pl.ds(sid * NSL, NSL)],
        part_hbm.at[cid, pl.ds(sid * NSL, NSL)],
    )


def _dinv_body(degp_ref, o_ref):
    d = degp_ref[0] + degp_ref[1]
    d = jnp.where(d < 0.5, d + 1.0, d)
    o_ref[:, :] = lax.rsqrt(d)


def _comb_body(coef_ref, p_ref, y_ref, w_ref, o_ref):
    a = coef_ref[0]
    b = coef_ref[1]
    c = coef_ref[2]
    o_ref[:, :] = a * (p_ref[0] + p_ref[1]) + b * y_ref[:, :] + c * w_ref[:, :]


_COMB_ROWS = 512


def _combine(coef, part, y, w):
    grid = NP // _COMB_ROWS
    return pl.pallas_call(
        _comb_body,
        grid=(grid,),
        in_specs=[
            pl.BlockSpec(memory_space=pltpu.SMEM),
            pl.BlockSpec((NC, _COMB_ROWS, D), lambda i: (0, i, 0)),
            pl.BlockSpec((_COMB_ROWS, D), lambda i: (i, 0)),
            pl.BlockSpec((_COMB_ROWS, D), lambda i: (i, 0)),
        ],
        out_specs=pl.BlockSpec((_COMB_ROWS, D), lambda i: (i, 0)),
        out_shape=jax.ShapeDtypeStruct((NP, D), jnp.float32),
    )(coef, part, y, w)


def kernel(x, edge_index, edge_attr, alphas):
    a_, b_, lo, hi = 1.0, 1.0, -1.0, 1.0
    al = jnp.tanh(alphas)
    coefs = []
    c1 = ((a_ - b_) / 2 - (a_ + b_ + 2) / 2 * (lo + hi) / (hi - lo)) * al[0]
    c2 = ((a_ + b_ + 2) / (hi - lo)) * al[0]
    coefs.append(jnp.stack([c2, c1, jnp.zeros(())]).astype(jnp.float32))
    for L in range(2, DEPTH + 1):
        coef_l = 2 * L * (L + a_ + b_) * (2 * L - 2 + a_ + b_)
        coef_lm1_1 = (2 * L + a_ + b_ - 1) * (2 * L + a_ + b_) * (2 * L + a_ + b_ - 2)
        coef_lm1_2 = (2 * L + a_ + b_ - 1) * (a_**2 - b_**2)
        coef_lm2 = 2 * (L - 1 + a_) * (L - 1 + b_) * (2 * L + a_ + b_)
        tmp1 = al[L - 1] * (coef_lm1_1 / coef_l)
        tmp2 = al[L - 1] * (coef_lm1_2 / coef_l)
        tmp3 = al[L - 1] * al[L - 2] * (coef_lm2 / coef_l)
        tmp1_2 = tmp1 * (2 / (hi - lo))
        tmp2_2 = tmp1 * ((hi + lo) / (hi - lo)) + tmp2
        coefs.append(jnp.stack([tmp1_2, -tmp2_2, -tmp3]).astype(jnp.float32))

    row = edge_index[0]
    col = edge_index[1]
    pad = EP - E
    row_t = jnp.concatenate([row, jnp.full((pad,), N_NODES, jnp.int32)]).reshape(
        NW, CH, 128
    )
    col_t = jnp.concatenate([col, jnp.full((pad,), N_NODES, jnp.int32)]).reshape(
        NW, CH, 128
    )
    attr_t = jnp.concatenate(
        [edge_attr, jnp.zeros((pad,), jnp.float32)]
    ).reshape(NW, CH, 128)
    xp = jnp.pad(x, ((0, NP - N_NODES), (0, 0)))

    degp = _deg_kernel(row_t)
    dinv2 = pl.pallas_call(
        _dinv_body,
        out_shape=jax.ShapeDtypeStruct((NP // 128, 128), jnp.float32),
    )(degp.reshape(NC, NP // 128, 128))
    dinv = dinv2.reshape(NP)
    val_t = _val_kernel(row_t, col_t, attr_t, dinv)

    row_t5 = row_t.reshape(NW, EPT)
    col_t5 = col_t.reshape(NW, EPT)
    val_t5 = val_t.reshape(NW, EPT)
    ys = [xp]
    for L in range(1, DEPTH + 1):
        part = _spmm_kernel(ys[-1], row_t5, col_t5, val_t5)
        w = ys[-2] if L >= 2 else xp
        ys.append(_combine(coefs[L - 1], part, ys[-1], w))

    return jnp.stack(ys, axis=1)[:N_NODES]


# speedup vs baseline: 1.1515x; 1.1515x over previous
import functools

import jax
import jax.numpy as jnp
from jax import lax
from jax.experimental import pallas as pl
from jax.experimental.pallas import tpu as pltpu
from jax.experimental.pallas import tpu_sc as plsc

N_NODES = 10000
D = 128
E = 320000
DEPTH = 3
NC = 2
NS = 16
NW = NC * NS
SB = 2560
EPT0 = 6 * SB
EPT1 = 2 * SB
EPTMAX = EPT0
CHM = EPTMAX // 128
NP = 10240
NSL = NP // NS
GC = 128
CPS = SB // GC

_mesh = plsc.VectorSubcoreMesh(
    core_axis_name="c", subcore_axis_name="s", num_cores=NC, num_subcores=NS
)


def _zero_vmem_2d(ref, rows):
    zero16 = jnp.zeros((16,), jnp.float32)

    def body(r, carry):
        for k in range(8):
            ref[r, pl.ds(k * 16, 16)] = zero16
        return carry

    lax.fori_loop(0, rows, body, 0)


def _zero_vmem_1d(ref, n):
    zero16 = jnp.zeros((16,), jnp.float32)

    def body(i, carry):
        ref[pl.ds(i * 16, 16)] = zero16
        return carry

    lax.fori_loop(0, n // 16, body, 0)


@functools.partial(
    pl.kernel,
    out_type=jax.ShapeDtypeStruct((NC, NP), jnp.float32),
    mesh=_mesh,
    compiler_params=pltpu.CompilerParams(needs_layout_passes=False),
    scratch_types=[
        pltpu.VMEM((CHM, 128), jnp.int32),
        pltpu.VMEM((128,), jnp.float32),
        pltpu.VMEM((NSL,), jnp.float32),
        pltpu.VMEM_SHARED((NP,), jnp.float32),
    ],
)
def _deg_kernel(row_hbm, degp_hbm, row_v, ones_v, zrow_v, deg_sh):
    cid = lax.axis_index("c")
    sid = lax.axis_index("s")
    wid = cid * NS + sid
    pltpu.sync_copy(row_hbm.at[wid], row_v)
    one16 = jnp.ones((16,), jnp.float32)
    for k in range(8):
        ones_v[pl.ds(k * 16, 16)] = one16
    _zero_vmem_1d(zrow_v, NSL)
    pltpu.sync_copy(zrow_v, deg_sh.at[pl.ds(sid * NSL, NSL)])
    plsc.subcore_barrier()

    def body(j, carry):
        pltpu.sync_copy(ones_v, deg_sh.at[row_v.at[j]], add=True)
        return carry

    lax.fori_loop(0, CHM, body, 0)
    plsc.subcore_barrier()
    pltpu.sync_copy(
        deg_sh.at[pl.ds(sid * NSL, NSL)], degp_hbm.at[cid, pl.ds(sid * NSL, NSL)]
    )


@functools.partial(
    pl.kernel,
    out_type=jax.ShapeDtypeStruct((NW, CHM, 128), jnp.float32),
    mesh=_mesh,
    compiler_params=pltpu.CompilerParams(needs_layout_passes=False),
    scratch_types=[
        pltpu.VMEM((CHM, 128), jnp.int32),
        pltpu.VMEM((CHM, 128), jnp.int32),
        pltpu.VMEM((CHM, 128), jnp.float32),
        pltpu.VMEM((NP,), jnp.float32),
    ],
)
def _val_kernel(row_hbm, col_hbm, attr_hbm, dinv_hbm, val_hbm,
                row_v, col_v, val_v, dinv_v):
    cid = lax.axis_index("c")
    sid = lax.axis_index("s")
    wid = cid * NS + sid
    pltpu.sync_copy(row_hbm.at[wid], row_v)
    pltpu.sync_copy(col_hbm.at[wid], col_v)
    pltpu.sync_copy(attr_hbm.at[wid], val_v)
    pltpu.sync_copy(dinv_hbm, dinv_v)

    def val_body(j, carry):
        for k in range(8):
            s = pl.ds(k * 16, 16)
            r16 = row_v[j, s]
            c16 = col_v[j, s]
            a16 = val_v[j, s]
            dr = plsc.load_gather(dinv_v, [r16])
            dc = plsc.load_gather(dinv_v, [c16])
            val_v[j, s] = dr * a16 * dc
        return carry

    lax.fori_loop(0, CHM, val_body, 0)
    pltpu.sync_copy(val_v, val_hbm.at[wid])


@functools.partial(
    pl.kernel,
    out_type=jax.ShapeDtypeStruct((NC, NP, D), jnp.float32),
    mesh=_mesh,
    compiler_params=pltpu.CompilerParams(needs_layout_passes=False),
    scratch_types=[
        pltpu.VMEM((SB,), jnp.int32),
        pltpu.VMEM((SB,), jnp.int32),
        pltpu.VMEM((SB,), jnp.float32),
        pltpu.VMEM((2, GC, D), jnp.float32),
        pltpu.VMEM_SHARED((NP, D), jnp.float32),
        pltpu.SemaphoreType.DMA,
        pltpu.SemaphoreType.DMA,
    ],
)
def _spmm_kernel(
    y_hbm, row_hbm, col_hbm, valw_hbm, part_hbm,
    row_m, col_m, val_m, g2, acc_sh, sem_g, sem_s
):
    cid = lax.axis_index("c")
    sid = lax.axis_index("s")
    wid = cid * NS + sid

    _zero_vmem_2d(g2.at[0], GC)
    for blk in range(NSL // GC):
        pltpu.sync_copy(g2.at[0], acc_sh.at[pl.ds(sid * NSL + blk * GC, GC)])
    plsc.subcore_barrier()

    def gather_desc(k, b):
        return pltpu.make_async_copy(
            y_hbm.at[col_m.at[pl.ds(k * GC, GC)]], g2.at[b], sem_g
        )

    def scatter_wait(k, b):
        pltpu.make_async_copy(
            g2.at[b], acc_sh.at[row_m.at[pl.ds(k * GC, GC)]], sem_s
        ).wait()

    hi_sb = jnp.where(cid == 0, EPT0 // SB, EPT1 // SB)

    def sb_body(sb, carry):
        base = sb * SB
        pltpu.sync_copy(row_hbm.at[wid, pl.ds(base, SB)], row_m)
        pltpu.sync_copy(col_hbm.at[wid, pl.ds(base, SB)], col_m)
        pltpu.sync_copy(valw_hbm.at[wid, pl.ds(base, SB)], val_m)

        gather_desc(0, 0).start()

        def chunk_body(k, c2):
            b = k % 2
            nb = (k + 1) % 2
            gather_desc(k, b).wait()

            def scale_body(e0, c3):
                vchunk = val_m[pl.ds(k * GC + e0 * 16, 16)]
                for l in range(16):
                    v = vchunk[l]
                    e = e0 * 16 + l
                    for kk in range(8):
                        s = pl.ds(kk * 16, 16)
                        g2[b, e, s] = g2[b, e, s] * v
                return c3

            lax.fori_loop(0, GC // 16, scale_body, 0)
            pltpu.async_copy(
                g2.at[b], acc_sh.at[row_m.at[pl.ds(k * GC, GC)]], sem_s, add=True
            )

            @pl.when(jnp.logical_and(k >= 1, k + 1 < CPS))
            def _():
                scatter_wait(k - 1, nb)

            @pl.when(k + 1 < CPS)
            def _():
                gather_desc(k + 1, nb).start()

            return c2

        lax.fori_loop(0, CPS, chunk_body, 0)
        scatter_wait(CPS - 2, (CPS - 2) % 2)
        scatter_wait(CPS - 1, (CPS - 1) % 2)
        return carry

    lax.fori_loop(0, hi_sb, sb_body, 0)
    plsc.subcore_barrier()
    pltpu.sync_copy(
        acc_sh.at[pl.ds(sid * NSL, NSL)],
        part_hbm.at[cid, pl.ds(sid * NSL, NSL)],
    )


def _dinv_body(degp_ref, o_ref):
    d = degp_ref[0] + degp_ref[1]
    d = jnp.where(d < 0.5, d + 1.0, d)
    o_ref[:, :] = lax.rsqrt(d)


def _comb_body(coef_ref, p_ref, y_ref, w_ref, o_ref):
    a = coef_ref[0]
    b = coef_ref[1]
    c = coef_ref[2]
    o_ref[:, :] = a * (p_ref[0] + p_ref[1]) + b * y_ref[:, :] + c * w_ref[:, :]


_COMB_ROWS = 512


def _combine(coef, part, y, w):
    grid = NP // _COMB_ROWS
    return pl.pallas_call(
        _comb_body,
        grid=(grid,),
        in_specs=[
            pl.BlockSpec(memory_space=pltpu.SMEM),
            pl.BlockSpec((NC, _COMB_ROWS, D), lambda i: (0, i, 0)),
            pl.BlockSpec((_COMB_ROWS, D), lambda i: (i, 0)),
            pl.BlockSpec((_COMB_ROWS, D), lambda i: (i, 0)),
        ],
        out_specs=pl.BlockSpec((_COMB_ROWS, D), lambda i: (i, 0)),
        out_shape=jax.ShapeDtypeStruct((NP, D), jnp.float32),
    )(coef, part, y, w)


def kernel(x, edge_index, edge_attr, alphas):
    a_, b_, lo, hi = 1.0, 1.0, -1.0, 1.0
    al = jnp.tanh(alphas)
    coefs = []
    c1 = ((a_ - b_) / 2 - (a_ + b_ + 2) / 2 * (lo + hi) / (hi - lo)) * al[0]
    c2 = ((a_ + b_ + 2) / (hi - lo)) * al[0]
    coefs.append(jnp.stack([c2, c1, jnp.zeros(())]).astype(jnp.float32))
    for L in range(2, DEPTH + 1):
        coef_l = 2 * L * (L + a_ + b_) * (2 * L - 2 + a_ + b_)
        coef_lm1_1 = (2 * L + a_ + b_ - 1) * (2 * L + a_ + b_) * (2 * L + a_ + b_ - 2)
        coef_lm1_2 = (2 * L + a_ + b_ - 1) * (a_**2 - b_**2)
        coef_lm2 = 2 * (L - 1 + a_) * (L - 1 + b_) * (2 * L + a_ + b_)
        tmp1 = al[L - 1] * (coef_lm1_1 / coef_l)
        tmp2 = al[L - 1] * (coef_lm1_2 / coef_l)
        tmp3 = al[L - 1] * al[L - 2] * (coef_lm2 / coef_l)
        tmp1_2 = tmp1 * (2 / (hi - lo))
        tmp2_2 = tmp1 * ((hi + lo) / (hi - lo)) + tmp2
        coefs.append(jnp.stack([tmp1_2, -tmp2_2, -tmp3]).astype(jnp.float32))

    def lay_out(arr, fill):
        n_a = NS * EPT0
        part_a = arr[:n_a].reshape(NS, EPT0)
        tail = jnp.full((NS * EPT1 - (E - n_a),), fill, arr.dtype)
        part_b = jnp.concatenate([arr[n_a:], tail]).reshape(NS, EPT1)
        part_b = jnp.pad(part_b, ((0, 0), (0, EPTMAX - EPT1)),
                         constant_values=fill)
        return jnp.concatenate([part_a, part_b], axis=0)

    row_h = lay_out(edge_index[0], N_NODES)
    col_h = lay_out(edge_index[1], N_NODES)
    attr_h = lay_out(edge_attr, 0.0)
    row_t = row_h.reshape(NW, CHM, 128)
    col_t = col_h.reshape(NW, CHM, 128)
    attr_t = attr_h.reshape(NW, CHM, 128)
    xp = jnp.pad(x, ((0, NP - N_NODES), (0, 0)))

    degp = _deg_kernel(row_t)
    dinv2 = pl.pallas_call(
        _dinv_body,
        out_shape=jax.ShapeDtypeStruct((NP // 128, 128), jnp.float32),
    )(degp.reshape(NC, NP // 128, 128))
    dinv = dinv2.reshape(NP)
    val_t = _val_kernel(row_t, col_t, attr_t, dinv)

    val_h = val_t.reshape(NW, EPTMAX)
    ys = [xp]
    for L in range(1, DEPTH + 1):
        part = _spmm_kernel(ys[-1], row_h, col_h, val_h)
        w = ys[-2] if L >= 2 else xp
        ys.append(_combine(coefs[L - 1], part, ys[-1], w))

    return jnp.stack(ys, axis=1)[:N_NODES]


# speedup vs baseline: 1.8211x; 1.5815x over previous
import functools

import jax
import jax.numpy as jnp
from jax import lax
from jax.experimental import pallas as pl
from jax.experimental.pallas import tpu as pltpu
from jax.experimental.pallas import tpu_sc as plsc

N_NODES = 10000
D = 128
E = 320000
DEPTH = 3
NC = 2
NS = 16
NW = NC * NS
SB = 2560
EPT0 = 6 * SB
EPT1 = 2 * SB
EPTMAX = EPT0
CHM = EPTMAX // 128
NP = 10240
NSL = NP // NS
GC = 128
CPS = SB // GC

_mesh = plsc.VectorSubcoreMesh(
    core_axis_name="c", subcore_axis_name="s", num_cores=NC, num_subcores=NS
)


def _zero_vmem_2d(ref, rows):
    zero16 = jnp.zeros((16,), jnp.float32)

    def body(r, carry):
        for k in range(8):
            ref[r, pl.ds(k * 16, 16)] = zero16
        return carry

    lax.fori_loop(0, rows, body, 0)


def _zero_vmem_1d(ref, n):
    zero16 = jnp.zeros((16,), jnp.float32)

    def body(i, carry):
        ref[pl.ds(i * 16, 16)] = zero16
        return carry

    lax.fori_loop(0, n // 16, body, 0)


@functools.partial(
    pl.kernel,
    out_type=jax.ShapeDtypeStruct((NC, NP), jnp.float32),
    mesh=_mesh,
    compiler_params=pltpu.CompilerParams(needs_layout_passes=False),
    scratch_types=[
        pltpu.VMEM((CHM, 128), jnp.int32),
        pltpu.VMEM((128,), jnp.float32),
        pltpu.VMEM((NSL,), jnp.float32),
        pltpu.VMEM_SHARED((NP,), jnp.float32),
    ],
)
def _deg_kernel(row_hbm, degp_hbm, row_v, ones_v, zrow_v, deg_sh):
    cid = lax.axis_index("c")
    sid = lax.axis_index("s")
    wid = cid * NS + sid
    pltpu.sync_copy(row_hbm.at[wid], row_v)
    one16 = jnp.ones((16,), jnp.float32)
    for k in range(8):
        ones_v[pl.ds(k * 16, 16)] = one16
    _zero_vmem_1d(zrow_v, NSL)
    pltpu.sync_copy(zrow_v, deg_sh.at[pl.ds(sid * NSL, NSL)])
    plsc.subcore_barrier()

    def body(j, carry):
        pltpu.sync_copy(ones_v, deg_sh.at[row_v.at[j]], add=True)
        return carry

    lax.fori_loop(0, CHM, body, 0)
    plsc.subcore_barrier()
    pltpu.sync_copy(
        deg_sh.at[pl.ds(sid * NSL, NSL)], degp_hbm.at[cid, pl.ds(sid * NSL, NSL)]
    )


@functools.partial(
    pl.kernel,
    out_type=jax.ShapeDtypeStruct((NW, CHM, 128), jnp.float32),
    mesh=_mesh,
    compiler_params=pltpu.CompilerParams(needs_layout_passes=False),
    scratch_types=[
        pltpu.VMEM((CHM, 128), jnp.int32),
        pltpu.VMEM((CHM, 128), jnp.int32),
        pltpu.VMEM((CHM, 128), jnp.float32),
        pltpu.VMEM((NP,), jnp.float32),
    ],
)
def _val_kernel(row_hbm, col_hbm, attr_hbm, dinv_hbm, val_hbm,
                row_v, col_v, val_v, dinv_v):
    cid = lax.axis_index("c")
    sid = lax.axis_index("s")
    wid = cid * NS + sid
    pltpu.sync_copy(row_hbm.at[wid], row_v)
    pltpu.sync_copy(col_hbm.at[wid], col_v)
    pltpu.sync_copy(attr_hbm.at[wid], val_v)
    pltpu.sync_copy(dinv_hbm, dinv_v)

    def val_body(j, carry):
        for k in range(8):
            s = pl.ds(k * 16, 16)
            r16 = row_v[j, s]
            c16 = col_v[j, s]
            a16 = val_v[j, s]
            dr = plsc.load_gather(dinv_v, [r16])
            dc = plsc.load_gather(dinv_v, [c16])
            val_v[j, s] = dr * a16 * dc
        return carry

    lax.fori_loop(0, CHM, val_body, 0)
    pltpu.sync_copy(val_v, val_hbm.at[wid])


@functools.partial(
    pl.kernel,
    out_type=jax.ShapeDtypeStruct((NC, NP, D), jnp.float32),
    mesh=_mesh,
    compiler_params=pltpu.CompilerParams(needs_layout_passes=False),
    scratch_types=[
        pltpu.VMEM((SB,), jnp.int32),
        pltpu.VMEM((SB,), jnp.int32),
        pltpu.VMEM((SB,), jnp.float32),
        pltpu.VMEM((2, GC, D), jnp.float32),
        pltpu.VMEM_SHARED((NP, D), jnp.float32),
        pltpu.SemaphoreType.DMA,
        pltpu.SemaphoreType.DMA,
    ],
)
def _spmm_kernel(
    y_hbm, row_hbm, col_hbm, valw_hbm, part_hbm,
    row_m, col_m, val_m, g2, acc_sh, sem_g, sem_s
):
    cid = lax.axis_index("c")
    sid = lax.axis_index("s")
    wid = cid * NS + sid

    _zero_vmem_2d(g2.at[0], GC)
    for blk in range(NSL // GC):
        pltpu.sync_copy(g2.at[0], acc_sh.at[pl.ds(sid * NSL + blk * GC, GC)])
    plsc.subcore_barrier()

    def gather_desc(k, b):
        return pltpu.make_async_copy(
            y_hbm.at[col_m.at[pl.ds(k * GC, GC)]], g2.at[b], sem_g
        )

    def scatter_wait(k, b):
        pltpu.make_async_copy(
            g2.at[b], acc_sh.at[row_m.at[pl.ds(k * GC, GC)]], sem_s
        ).wait()

    hi_sb = jnp.where(cid == 0, EPT0 // SB, EPT1 // SB)

    def sb_body(sb, carry):
        base = sb * SB
        pltpu.sync_copy(row_hbm.at[wid, pl.ds(base, SB)], row_m)
        pltpu.sync_copy(col_hbm.at[wid, pl.ds(base, SB)], col_m)
        pltpu.sync_copy(valw_hbm.at[wid, pl.ds(base, SB)], val_m)

        def chunk_body(k, c2):
            pltpu.sync_copy(y_hbm.at[col_m.at[pl.ds(k * GC, GC)]], g2.at[0])

            def scale_body(e0, c3):
                vchunk = val_m[pl.ds(k * GC + e0 * 16, 16)]
                for l in range(16):
                    v = vchunk[l]
                    e = e0 * 16 + l
                    for kk in range(8):
                        s = pl.ds(kk * 16, 16)
                        g2[0, e, s] = g2[0, e, s] * v
                return c3

            lax.fori_loop(0, GC // 16, scale_body, 0)
            pltpu.sync_copy(
                g2.at[0], acc_sh.at[row_m.at[pl.ds(k * GC, GC)]], add=True
            )
            return c2

        lax.fori_loop(0, CPS, chunk_body, 0)
        return carry

    lax.fori_loop(0, hi_sb, sb_body, 0)
    plsc.subcore_barrier()
    pltpu.sync_copy(
        acc_sh.at[pl.ds(sid * NSL, NSL)],
        part_hbm.at[cid, pl.ds(sid * NSL, NSL)],
    )


def _dinv_body(degp_ref, o_ref):
    d = degp_ref[0] + degp_ref[1]
    d = jnp.where(d < 0.5, d + 1.0, d)
    o_ref[:, :] = lax.rsqrt(d)


def _comb_body(coef_ref, p_ref, y_ref, w_ref, o_ref):
    a = coef_ref[0]
    b = coef_ref[1]
    c = coef_ref[2]
    o_ref[:, :] = a * (p_ref[0] + p_ref[1]) + b * y_ref[:, :] + c * w_ref[:, :]


_COMB_ROWS = 512


def _combine(coef, part, y, w):
    grid = NP // _COMB_ROWS
    return pl.pallas_call(
        _comb_body,
        grid=(grid,),
        in_specs=[
            pl.BlockSpec(memory_space=pltpu.SMEM),
            pl.BlockSpec((NC, _COMB_ROWS, D), lambda i: (0, i, 0)),
            pl.BlockSpec((_COMB_ROWS, D), lambda i: (i, 0)),
            pl.BlockSpec((_COMB_ROWS, D), lambda i: (i, 0)),
        ],
        out_specs=pl.BlockSpec((_COMB_ROWS, D), lambda i: (i, 0)),
        out_shape=jax.ShapeDtypeStruct((NP, D), jnp.float32),
    )(coef, part, y, w)


def kernel(x, edge_index, edge_attr, alphas):
    a_, b_, lo, hi = 1.0, 1.0, -1.0, 1.0
    al = jnp.tanh(alphas)
    coefs = []
    c1 = ((a_ - b_) / 2 - (a_ + b_ + 2) / 2 * (lo + hi) / (hi - lo)) * al[0]
    c2 = ((a_ + b_ + 2) / (hi - lo)) * al[0]
    coefs.append(jnp.stack([c2, c1, jnp.zeros(())]).astype(jnp.float32))
    for L in range(2, DEPTH + 1):
        coef_l = 2 * L * (L + a_ + b_) * (2 * L - 2 + a_ + b_)
        coef_lm1_1 = (2 * L + a_ + b_ - 1) * (2 * L + a_ + b_) * (2 * L + a_ + b_ - 2)
        coef_lm1_2 = (2 * L + a_ + b_ - 1) * (a_**2 - b_**2)
        coef_lm2 = 2 * (L - 1 + a_) * (L - 1 + b_) * (2 * L + a_ + b_)
        tmp1 = al[L - 1] * (coef_lm1_1 / coef_l)
        tmp2 = al[L - 1] * (coef_lm1_2 / coef_l)
        tmp3 = al[L - 1] * al[L - 2] * (coef_lm2 / coef_l)
        tmp1_2 = tmp1 * (2 / (hi - lo))
        tmp2_2 = tmp1 * ((hi + lo) / (hi - lo)) + tmp2
        coefs.append(jnp.stack([tmp1_2, -tmp2_2, -tmp3]).astype(jnp.float32))

    def lay_out(arr, fill):
        n_a = NS * EPT0
        part_a = arr[:n_a].reshape(NS, EPT0)
        tail = jnp.full((NS * EPT1 - (E - n_a),), fill, arr.dtype)
        part_b = jnp.concatenate([arr[n_a:], tail]).reshape(NS, EPT1)
        part_b = jnp.pad(part_b, ((0, 0), (0, EPTMAX - EPT1)),
                         constant_values=fill)
        return jnp.concatenate([part_a, part_b], axis=0)

    row_h = lay_out(edge_index[0], N_NODES)
    col_h = lay_out(edge_index[1], N_NODES)
    attr_h = lay_out(edge_attr, 0.0)
    row_t = row_h.reshape(NW, CHM, 128)
    col_t = col_h.reshape(NW, CHM, 128)
    attr_t = attr_h.reshape(NW, CHM, 128)
    xp = jnp.pad(x, ((0, NP - N_NODES), (0, 0)))

    degp = _deg_kernel(row_t)
    dinv2 = pl.pallas_call(
        _dinv_body,
        out_shape=jax.ShapeDtypeStruct((NP // 128, 128), jnp.float32),
    )(degp.reshape(NC, NP // 128, 128))
    dinv = dinv2.reshape(NP)
    val_t = _val_kernel(row_t, col_t, attr_t, dinv)

    val_h = val_t.reshape(NW, EPTMAX)
    ys = [xp]
    for L in range(1, DEPTH + 1):
        part = _spmm_kernel(ys[-1], row_h, col_h, val_h)
        w = ys[-2] if L >= 2 else xp
        ys.append(_combine(coefs[L - 1], part, ys[-1], w))

    return jnp.stack(ys, axis=1)[:N_NODES]
